# baseline (speedup 1.0000x reference)
import functools

import jax
import jax.numpy as jnp
from jax.experimental import pallas as pl
from jax.experimental.pallas import tpu as pltpu


def _entropy(means):
    zeromask = (means == 0).astype(jnp.float32)
    onesmask = (means == 1).astype(jnp.float32)
    safemean = 1e-08 * zeromask - 1e-08 * onesmask + means
    return -safemean * jnp.log2(safemean) - (1.0 - safemean) * jnp.log2(1.0 - safemean)


def _fused_kernel(n_match, addr_ref, loc_ref, cnt_ref, means_ref, radius_ref,
                  out_ref, aw_ref, thr_ref, act_ref, acc_ref):
    j = pl.program_id(0)

    @pl.when(j == 0)
    def _():
        w = _entropy(means_ref[...])
        a = addr_ref[...].astype(jnp.float32)
        aw_ref[...] = (w - 2.0 * (w * a)).astype(jnp.bfloat16)
        thr_ref[...] = radius_ref[0] - jnp.sum(w * a, axis=1, keepdims=True)
        act_ref[1] = jnp.zeros_like(act_ref[1])
        acc_ref[...] = jnp.zeros_like(acc_ref)

    partial = jax.lax.dot_general(
        act_ref[(j + 1) % 2], cnt_ref[...].astype(jnp.bfloat16),
        (((1,), (0,)), ((), ())),
        preferred_element_type=jnp.float32)
    acc_ref[...] += partial

    cross = jax.lax.dot_general(
        aw_ref[...], loc_ref[...].astype(jnp.bfloat16),
        (((1,), (1,)), ((), ())),
        preferred_element_type=jnp.float32)
    act_ref[j % 2] = (cross <= thr_ref[...]).astype(jnp.bfloat16)

    @pl.when(j == n_match)
    def _():
        out_ref[...] = (acc_ref[...] > 0).astype(jnp.uint8)


@jax.jit
def kernel(address, locations, counter, means, radius):
    B, A = address.shape
    _, N, M = counter.shape
    loc2d = locations.reshape(N, A)
    cnt2d = counter.reshape(N, M)
    means2d = means.reshape(1, A)
    radius_arr = jnp.asarray(radius, jnp.float32).reshape(1)

    BN = 1024
    n_match = N // BN
    grid = (n_match + 1,)

    out = pl.pallas_call(
        functools.partial(_fused_kernel, n_match),
        grid=grid,
        in_specs=[
            pl.BlockSpec((B, A), lambda j: (0, 0)),
            pl.BlockSpec((BN, A), lambda j: (jnp.minimum(j, n_match - 1), 0)),
            pl.BlockSpec((BN, M), lambda j: (jnp.maximum(j - 1, 0), 0)),
            pl.BlockSpec((1, A), lambda j: (0, 0)),
            pl.BlockSpec(memory_space=pltpu.SMEM),
        ],
        out_specs=pl.BlockSpec((B, M), lambda j: (0, 0)),
        out_shape=jax.ShapeDtypeStruct((B, M), jnp.uint8),
        scratch_shapes=[pltpu.VMEM((B, A), jnp.bfloat16),
                        pltpu.VMEM((B, 1), jnp.float32),
                        pltpu.VMEM((2, B, BN), jnp.bfloat16),
                        pltpu.VMEM((B, M), jnp.float32)],
        compiler_params=pltpu.CompilerParams(
            dimension_semantics=("arbitrary",)),
    )(address, loc2d, cnt2d, means2d, radius_arr)
    return out


# speedup vs baseline: 1.0788x; 1.0788x over previous
import functools

import jax
import jax.numpy as jnp
from jax.experimental import pallas as pl
from jax.experimental.pallas import tpu as pltpu


def _entropy(means):
    zeromask = (means == 0).astype(jnp.float32)
    onesmask = (means == 1).astype(jnp.float32)
    safemean = 1e-08 * zeromask - 1e-08 * onesmask + means
    return -safemean * jnp.log2(safemean) - (1.0 - safemean) * jnp.log2(1.0 - safemean)


def _fused_kernel(bn, n_steps, addr_ref, loc_ref, cnt_ref, means_ref,
                  radius_ref, out_ref, aw_ref, thr_ref, act_ref, cntb_ref,
                  acc_ref):
    j = pl.program_id(0)
    half = (n_steps // 2) * bn

    @pl.when(j == 0)
    def _():
        w = _entropy(means_ref[...])
        a = addr_ref[...].astype(jnp.float32)
        aw_ref[...] = (w - 2.0 * (w * a)).astype(jnp.bfloat16)
        thr_ref[...] = (radius_ref[0] -
                        jnp.sum(w * a, axis=1, keepdims=True))

    cross = jax.lax.dot_general(
        aw_ref[...], loc_ref[...].astype(jnp.bfloat16),
        (((1,), (1,)), ((), ())),
        preferred_element_type=jnp.float32)
    act_ref[:, pl.ds(j * bn, bn)] = (cross <= thr_ref[...]).astype(jnp.bfloat16)
    cntb_ref[pl.ds(j * bn, bn), :] = cnt_ref[...].astype(jnp.bfloat16)

    @pl.when(j == n_steps - 2)
    def _():
        acc_ref[...] = jax.lax.dot_general(
            act_ref[:, :half], cntb_ref[:half, :],
            (((1,), (0,)), ((), ())),
            preferred_element_type=jnp.float32)

    @pl.when(j == n_steps - 1)
    def _():
        acc = acc_ref[...] + jax.lax.dot_general(
            act_ref[:, half:], cntb_ref[half:, :],
            (((1,), (0,)), ((), ())),
            preferred_element_type=jnp.float32)
        out_ref[...] = (acc > 0).astype(jnp.uint8)


@jax.jit
def kernel(address, locations, counter, means, radius):
    B, A = address.shape
    _, N, M = counter.shape
    loc2d = locations.reshape(N, A)
    cnt2d = counter.reshape(N, M)
    means2d = means.reshape(1, A)
    radius_arr = jnp.asarray(radius, jnp.float32).reshape(1)

    BN = 1024
    n_steps = N // BN
    grid = (n_steps,)

    out = pl.pallas_call(
        functools.partial(_fused_kernel, BN, n_steps),
        grid=grid,
        in_specs=[
            pl.BlockSpec((B, A), lambda j: (0, 0)),
            pl.BlockSpec((BN, A), lambda j: (j, 0)),
            pl.BlockSpec((BN, M), lambda j: (j, 0)),
            pl.BlockSpec((1, A), lambda j: (0, 0)),
            pl.BlockSpec(memory_space=pltpu.SMEM),
        ],
        out_specs=pl.BlockSpec((B, M), lambda j: (0, 0)),
        out_shape=jax.ShapeDtypeStruct((B, M), jnp.uint8),
        scratch_shapes=[pltpu.VMEM((B, A), jnp.bfloat16),
                        pltpu.VMEM((B, 1), jnp.float32),
                        pltpu.VMEM((B, N), jnp.bfloat16),
                        pltpu.VMEM((N, M), jnp.bfloat16),
                        pltpu.VMEM((B, M), jnp.float32)],
        compiler_params=pltpu.CompilerParams(
            dimension_semantics=("arbitrary",)),
    )(address, loc2d, cnt2d, means2d, radius_arr)
    return out
